# baseline (speedup 1.0000x reference)
import jax
import jax.numpy as jnp
from jax.experimental import pallas as pl
from jax.experimental.pallas import tpu as pltpu

N_POINTS = 100000
C = 256
R = 8
NR = N_POINTS // R
BLKG = 2048


def _fused_body(idx_ref, tab_ref, bin_ref, coord_ref, w24_ref, wout_ref, bout_ref, out_ref):
    del idx_ref
    ctx = tab_ref[0] + bin_ref[...]
    h2 = jax.lax.dot_general(
        coord_ref[...],
        w24_ref[...],
        (((0,), (0,)), ((), ())),
        preferred_element_type=jnp.float32,
    )
    wout = wout_ref[...]
    bout = bout_ref[...]
    for a in range(R):
        h = jnp.maximum(h2[:, C * a : C * (a + 1)] + ctx, 0.0)
        out_ref[a] = jnp.dot(h, wout, preferred_element_type=jnp.float32) + bout


def kernel(coord, condition_idx, embedding_table, W_in, b_in, W_out, b_out):
    idx = condition_idx.astype(jnp.int32)
    coord24 = coord.T.reshape(3, R, NR).transpose(1, 0, 2).reshape(3 * R, NR)
    w24 = (jnp.eye(R, dtype=jnp.float32)[:, None, :, None]
           * W_in[None, :, None, :]).reshape(3 * R, R * C)
    grid_spec = pltpu.PrefetchScalarGridSpec(
        num_scalar_prefetch=1,
        grid=((NR + BLKG - 1) // BLKG,),
        in_specs=[
            pl.BlockSpec((1, 1, C), lambda i, idx: (idx[0], 0, 0)),
            pl.BlockSpec((1, C), lambda i, idx: (0, 0)),
            pl.BlockSpec((3 * R, BLKG), lambda i, idx: (0, i)),
            pl.BlockSpec((3 * R, R * C), lambda i, idx: (0, 0)),
            pl.BlockSpec((C, C), lambda i, idx: (0, 0)),
            pl.BlockSpec((1, C), lambda i, idx: (0, 0)),
        ],
        out_specs=pl.BlockSpec((R, BLKG, C), lambda i, idx: (0, i, 0)),
    )
    out = pl.pallas_call(
        _fused_body,
        grid_spec=grid_spec,
        out_shape=jax.ShapeDtypeStruct((R, NR, C), jnp.float32),
        compiler_params=pltpu.CompilerParams(dimension_semantics=("arbitrary",)),
    )(
        idx,
        embedding_table.reshape(3, 1, C),
        b_in.reshape(1, C),
        coord24,
        w24,
        W_out,
        b_out.reshape(1, C),
    )
    return out.reshape(N_POINTS, C)


# speedup vs baseline: 2.8127x; 2.8127x over previous
import jax
import jax.numpy as jnp
from jax.experimental import pallas as pl
from jax.experimental.pallas import tpu as pltpu

N_POINTS = 100000
C = 256
R = 5
NR = N_POINTS // R
BLKG = 2560


def _fused_body(idx_ref, tab_ref, bin_ref, coord_ref, w24_ref, wout_ref, bout_ref, out_ref):
    del idx_ref
    ctx = tab_ref[0] + bin_ref[...]
    h2 = jax.lax.dot_general(
        coord_ref[...],
        w24_ref[...],
        (((0,), (0,)), ((), ())),
        preferred_element_type=jnp.float32,
    )
    wout = wout_ref[...]
    bout = bout_ref[...]
    for a in range(R):
        h = jnp.maximum(h2[:, C * a : C * (a + 1)] + ctx, 0.0)
        out_ref[a] = jnp.dot(h, wout, preferred_element_type=jnp.float32) + bout


def kernel(coord, condition_idx, embedding_table, W_in, b_in, W_out, b_out):
    idx = condition_idx.astype(jnp.int32)
    coord24 = coord.T.reshape(3, R, NR).transpose(1, 0, 2).reshape(3 * R, NR)
    w24 = (jnp.eye(R, dtype=jnp.float32)[:, None, :, None]
           * W_in[None, :, None, :]).reshape(3 * R, R * C)
    grid_spec = pltpu.PrefetchScalarGridSpec(
        num_scalar_prefetch=1,
        grid=((NR + BLKG - 1) // BLKG,),
        in_specs=[
            pl.BlockSpec((1, 1, C), lambda i, idx: (idx[0], 0, 0)),
            pl.BlockSpec((1, C), lambda i, idx: (0, 0)),
            pl.BlockSpec((3 * R, BLKG), lambda i, idx: (0, i)),
            pl.BlockSpec((3 * R, R * C), lambda i, idx: (0, 0)),
            pl.BlockSpec((C, C), lambda i, idx: (0, 0)),
            pl.BlockSpec((1, C), lambda i, idx: (0, 0)),
        ],
        out_specs=pl.BlockSpec((R, BLKG, C), lambda i, idx: (0, i, 0)),
    )
    out = pl.pallas_call(
        _fused_body,
        grid_spec=grid_spec,
        out_shape=jax.ShapeDtypeStruct((R, NR, C), jnp.float32),
        compiler_params=pltpu.CompilerParams(dimension_semantics=("arbitrary",)),
    )(
        idx,
        embedding_table.reshape(3, 1, C),
        b_in.reshape(1, C),
        coord24,
        w24,
        W_out,
        b_out.reshape(1, C),
    )
    return out.reshape(N_POINTS, C)


# speedup vs baseline: 3.0680x; 1.0908x over previous
import jax
import jax.numpy as jnp
from jax.experimental import pallas as pl
from jax.experimental.pallas import tpu as pltpu

N_POINTS = 100000
C = 256
BLK = 16384


def _fused_body(idx_ref, tab_ref, bin_ref, coord_ref, win_ref, wout_ref, bout_ref, out_ref):
    del idx_ref
    ctx = tab_ref[0] + bin_ref[...]
    h = (
        jax.lax.dot_general(
            coord_ref[...],
            win_ref[...],
            (((0,), (0,)), ((), ())),
            preferred_element_type=jnp.float32,
        )
        + ctx
    )
    h = jnp.maximum(h, 0.0)
    out_ref[...] = (
        jnp.dot(h, wout_ref[...], preferred_element_type=jnp.float32) + bout_ref[...]
    )


def kernel(coord, condition_idx, embedding_table, W_in, b_in, W_out, b_out):
    idx = condition_idx.astype(jnp.int32)
    coord_t = coord.T
    grid_spec = pltpu.PrefetchScalarGridSpec(
        num_scalar_prefetch=1,
        grid=((N_POINTS + BLK - 1) // BLK,),
        in_specs=[
            pl.BlockSpec((1, 1, C), lambda i, idx: (idx[0], 0, 0)),
            pl.BlockSpec((1, C), lambda i, idx: (0, 0)),
            pl.BlockSpec((3, BLK), lambda i, idx: (0, i)),
            pl.BlockSpec((3, C), lambda i, idx: (0, 0)),
            pl.BlockSpec((C, C), lambda i, idx: (0, 0)),
            pl.BlockSpec((1, C), lambda i, idx: (0, 0)),
        ],
        out_specs=pl.BlockSpec((BLK, C), lambda i, idx: (i, 0)),
    )
    return pl.pallas_call(
        _fused_body,
        grid_spec=grid_spec,
        out_shape=jax.ShapeDtypeStruct((N_POINTS, C), jnp.float32),
        compiler_params=pltpu.CompilerParams(dimension_semantics=("arbitrary",)),
    )(
        idx,
        embedding_table.reshape(3, 1, C),
        b_in.reshape(1, C),
        coord_t,
        W_in,
        W_out,
        b_out.reshape(1, C),
    )
